# Initial kernel scaffold; baseline (speedup 1.0000x reference)
import jax
import jax.numpy as jnp
from jax.experimental import pallas as pl


def kernel(input, word_table, pos_table):
    raise NotImplementedError("write your pallas kernel here")


# speedup vs baseline: 4.4156x; 4.4156x over previous
import functools

import jax
import jax.numpy as jnp
from jax import lax
from jax.experimental import pallas as pl
from jax.experimental.pallas import tpu as pltpu
from jax.experimental.pallas import tpu_sc as plsc

NUM_CORES = 2
NUM_SUBCORES = 16
LANES = 16


def kernel(input, word_table, pos_table):
    B, S = input.shape
    V, E = word_table.shape
    NW = NUM_CORES * NUM_SUBCORES
    total = B * S
    per_w = total // NW
    n_seq = per_w // S
    half = S // 2
    idx2d = input.reshape(total // half, half)
    rows_per_w = 2 * n_seq

    mesh = plsc.VectorSubcoreMesh(core_axis_name="c", subcore_axis_name="s")

    @functools.partial(
        pl.kernel,
        out_type=jax.ShapeDtypeStruct((total, E), jnp.float32),
        mesh=mesh,
        scratch_types=[
            pltpu.VMEM((rows_per_w, half), jnp.int32),
            pltpu.VMEM((S, E), jnp.float32),
            pltpu.VMEM((S, E), jnp.float32),
            pltpu.SemaphoreType.DMA,
        ],
        compiler_params=pltpu.CompilerParams(use_tc_tiling_on_sc=False),
    )
    def sc_kernel(idx_hbm, word_hbm, pos_hbm, out_hbm, idx_v, pos_v, buf_v, sem):
        wid = lax.axis_index("s") * NUM_CORES + lax.axis_index("c")
        base = wid * per_w
        pltpu.sync_copy(idx_hbm.at[pl.ds(wid * rows_per_w, rows_per_w)], idx_v)
        pltpu.sync_copy(pos_hbm.at[pl.ds(0, S)], pos_v)

        def chunk_body(g, _):
            c0 = pltpu.async_copy(
                word_hbm.at[idx_v.at[2 * g]], buf_v.at[pl.ds(0, half)], sem)
            c1 = pltpu.async_copy(
                word_hbm.at[idx_v.at[2 * g + 1]], buf_v.at[pl.ds(half, half)], sem)
            c0.wait()
            c1.wait()

            def add_body(r, _):
                for c in range(E // LANES):
                    sl = pl.ds(c * LANES, LANES)
                    buf_v[r, sl] = buf_v[r, sl] + pos_v[r, sl]
                return 0

            lax.fori_loop(0, S, add_body, 0)
            pltpu.sync_copy(buf_v, out_hbm.at[pl.ds(base + g * S, S)])
            return 0

        lax.fori_loop(0, n_seq, chunk_body, 0)

    out = sc_kernel(idx2d, word_table, pos_table)
    return out.reshape(B, S, E)


# speedup vs baseline: 4.9700x; 1.1256x over previous
import functools

import jax
import jax.numpy as jnp
from jax import lax
from jax.experimental import pallas as pl
from jax.experimental.pallas import tpu as pltpu
from jax.experimental.pallas import tpu_sc as plsc

NUM_CORES = 2
NUM_SUBCORES = 16
LANES = 16


def kernel(input, word_table, pos_table):
    B, S = input.shape
    V, E = word_table.shape
    NW = NUM_CORES * NUM_SUBCORES
    total = B * S
    per_w = total // NW
    n_seq = per_w // S
    half = S // 2
    idx2d = input.reshape(total // half, half)
    rows_per_w = 2 * n_seq

    mesh = plsc.VectorSubcoreMesh(core_axis_name="c", subcore_axis_name="s")

    @functools.partial(
        pl.kernel,
        out_type=jax.ShapeDtypeStruct((total, E), jnp.float32),
        mesh=mesh,
        scratch_types=[
            pltpu.VMEM((rows_per_w, half), jnp.int32),
            pltpu.VMEM((S, E), jnp.float32),
            pltpu.VMEM((S, E), jnp.float32),
            pltpu.VMEM((S, E), jnp.float32),
            pltpu.SemaphoreType.DMA,
            pltpu.SemaphoreType.DMA,
            pltpu.SemaphoreType.DMA,
        ],
        compiler_params=pltpu.CompilerParams(use_tc_tiling_on_sc=False),
    )
    def sc_kernel(idx_hbm, word_hbm, pos_hbm, out_hbm,
                  idx_v, pos_v, buf0_v, buf1_v, gsem0, gsem1, ssem):
        wid = lax.axis_index("s") * NUM_CORES + lax.axis_index("c")
        base = wid * per_w
        bufs = (buf0_v, buf1_v)
        gsems = (gsem0, gsem1)
        pltpu.sync_copy(idx_hbm.at[pl.ds(wid * rows_per_w, rows_per_w)], idx_v)
        pltpu.sync_copy(pos_hbm.at[pl.ds(0, S)], pos_v)

        def fire_gather(g, slot):
            pltpu.async_copy(word_hbm.at[idx_v.at[2 * g]],
                             bufs[slot].at[pl.ds(0, half)], gsems[slot])
            pltpu.async_copy(word_hbm.at[idx_v.at[2 * g + 1]],
                             bufs[slot].at[pl.ds(half, half)], gsems[slot])

        def wait_gather(g, slot):
            pltpu.make_async_copy(word_hbm.at[idx_v.at[2 * g]],
                                  bufs[slot].at[pl.ds(0, half)],
                                  gsems[slot]).wait()
            pltpu.make_async_copy(word_hbm.at[idx_v.at[2 * g + 1]],
                                  bufs[slot].at[pl.ds(half, half)],
                                  gsems[slot]).wait()

        def wait_store(slot):
            pltpu.make_async_copy(bufs[slot], out_hbm.at[pl.ds(base, S)],
                                  ssem).wait()

        fire_gather(0, 0)

        def chunk_pair(gp, _):
            for b in range(2):
                g = gp * 2 + b

                if b == 0:
                    @pl.when(gp >= 1)
                    def _():
                        wait_store(1 - b)
                else:
                    wait_store(1 - b)

                @pl.when(g < n_seq - 1)
                def _():
                    fire_gather(g + 1, 1 - b)

                wait_gather(g, b)

                buf = bufs[b]

                @plsc.parallel_loop(0, S, unroll=8)
                def _(r):
                    for c in range(E // LANES):
                        sl = pl.ds(c * LANES, LANES)
                        buf[r, sl] = buf[r, sl] + pos_v[r, sl]

                pltpu.async_copy(buf, out_hbm.at[pl.ds(base + g * S, S)], ssem)
            return 0

        lax.fori_loop(0, n_seq // 2, chunk_pair, 0)
        wait_store(1)

    out = sc_kernel(idx2d, word_table, pos_table)
    return out.reshape(B, S, E)


# speedup vs baseline: 6.1489x; 1.2372x over previous
import functools

import jax
import jax.numpy as jnp
from jax import lax
from jax.experimental import pallas as pl
from jax.experimental.pallas import tpu as pltpu
from jax.experimental.pallas import tpu_sc as plsc

NUM_CORES = 2
NUM_SUBCORES = 16
LANES = 16


def kernel(input, word_table, pos_table):
    B, S = input.shape
    V, E = word_table.shape
    NW = NUM_CORES * NUM_SUBCORES
    total = B * S
    per_w = total // NW
    CH = 128
    n_ch = per_w // CH
    idx2d = input.T.reshape(total // CH, CH)
    idx_rows_w = per_w // CH
    pos_s = pos_table[:S]

    mesh = plsc.VectorSubcoreMesh(core_axis_name="c", subcore_axis_name="s")

    @functools.partial(
        pl.kernel,
        out_type=jax.ShapeDtypeStruct((total, E), jnp.float32),
        mesh=mesh,
        scratch_types=[
            pltpu.VMEM((idx_rows_w, CH), jnp.int32),
            pltpu.VMEM((S, E), jnp.float32),
            pltpu.VMEM((CH, E), jnp.float32),
            pltpu.VMEM((CH, E), jnp.float32),
            pltpu.SemaphoreType.DMA,
            pltpu.SemaphoreType.DMA,
            pltpu.SemaphoreType.DMA,
        ],
        compiler_params=pltpu.CompilerParams(use_tc_tiling_on_sc=False),
    )
    def sc_kernel(idx_hbm, word_hbm, pos_hbm, out_hbm,
                  idx_v, pos_v, buf0_v, buf1_v, gsem0, gsem1, ssem):
        wid = lax.axis_index("s") * NUM_CORES + lax.axis_index("c")
        base = wid * per_w
        bufs = (buf0_v, buf1_v)
        gsems = (gsem0, gsem1)
        pltpu.sync_copy(idx_hbm.at[pl.ds(wid * idx_rows_w, idx_rows_w)], idx_v)
        pltpu.sync_copy(pos_hbm, pos_v)

        def fire_gather(g, slot):
            pltpu.async_copy(word_hbm.at[idx_v.at[g]], bufs[slot], gsems[slot])

        def wait_gather(g, slot):
            pltpu.make_async_copy(word_hbm.at[idx_v.at[g]], bufs[slot],
                                  gsems[slot]).wait()

        def wait_store(slot):
            pltpu.make_async_copy(bufs[slot], out_hbm.at[pl.ds(base, CH)],
                                  ssem).wait()

        fire_gather(0, 0)

        def chunk_pair(gp, _):
            for b in range(2):
                g = gp * 2 + b

                if b == 0:
                    @pl.when(gp >= 1)
                    def _():
                        wait_store(1 - b)
                else:
                    wait_store(1 - b)

                @pl.when(g < n_ch - 1)
                def _():
                    fire_gather(g + 1, 1 - b)

                wait_gather(g, b)

                buf = bufs[b]
                srow = (base + g * CH) // B
                pvec = [pos_v[srow, pl.ds(c * LANES, LANES)]
                        for c in range(E // LANES)]

                @plsc.parallel_loop(0, CH, unroll=8)
                def _(r):
                    for c in range(E // LANES):
                        sl = pl.ds(c * LANES, LANES)
                        buf[r, sl] = buf[r, sl] + pvec[c]

                pltpu.async_copy(buf, out_hbm.at[pl.ds(base + g * CH, CH)],
                                 ssem)
            return 0

        lax.fori_loop(0, n_ch // 2, chunk_pair, 0)
        wait_store(1)

    out = sc_kernel(idx2d, word_table, pos_s)
    return out.reshape(S, B, E).transpose(1, 0, 2)
